# baseline (speedup 1.0000x reference)
import functools

import jax
import jax.numpy as jnp
from jax import lax
from jax.experimental import pallas as pl
from jax.experimental.pallas import tpu as pltpu
from jax.experimental.pallas import tpu_sc as plsc

K = 8192
D = 256
BETA = 0.25
NT = 256
N = 8192

SC_NC = 2
SC_NS = 16
SC_NW = SC_NC * SC_NS
ROWS_PER_W = N // SC_NW
GCHUNK = 128


def _vq_body(i, n_steps, f, e, mm_store_ref, mm_read_ref, fsqp_ref, esq_ref,
             enc_ref, idx_ref, loss_ref, perp_ref, counts_ref,
             loss_acc_ref):
    mm_store_ref[:] = jnp.dot(
        f * (-2.0), e.T, preferred_element_type=jnp.float32)

    mm = mm_read_ref[:]
    f_sq = fsqp_ref[:]
    e_sq = esq_ref[:]
    d = (f_sq + e_sq) + mm

    m = jnp.min(d, axis=1, keepdims=True)
    iota = jax.lax.broadcasted_iota(jnp.int32, (NT, K), 1).astype(jnp.float32)
    t = jnp.where(d == m, iota, jnp.float32(K))
    idx_f = jnp.min(t, axis=1, keepdims=True)
    idx_ref[:] = idx_f.astype(jnp.int32)

    enc = (iota == idx_f).astype(jnp.float32)
    enc_ref[:] = enc

    @pl.when(i <= 1)
    def _():
        counts_ref[:] = jnp.zeros_like(counts_ref)
        loss_acc_ref[0] = 0.0

    ones = jnp.ones((8, NT), jnp.float32)
    counts_ref[:] += jnp.dot(ones, enc,
                             preferred_element_type=jnp.float32)[0:1, :]
    loss_acc_ref[0] += jnp.sum(m)

    @pl.when(i == n_steps - 1)
    def _():
        p = counts_ref[0, :] * (1.0 / N)
        perp = jnp.exp(-jnp.sum(p * jnp.log(p + 1e-10)))
        perp_ref[:] = perp.reshape(1, 1)
        loss_ref[:] = ((1.0 + BETA) * loss_acc_ref[0] / (N * D)).reshape(1, 1)


def _vq_kernel(flat_ref, emb_ref, fsqp_ref, esq_ref, enc_ref, idx_ref,
               loss_ref, perp_ref, mma_ref, mmb_ref, counts_ref,
               loss_acc_ref):
    i = pl.program_id(0)
    n_steps = pl.num_programs(0)
    f = flat_ref[:]
    e = emb_ref[:]

    common = (fsqp_ref, esq_ref, enc_ref, idx_ref, loss_ref, perp_ref,
              counts_ref, loss_acc_ref)

    @pl.when(lax.rem(i, 2) == 0)
    def _():
        _vq_body(i, n_steps, f, e, mma_ref, mmb_ref, *common)

    @pl.when(lax.rem(i, 2) == 1)
    def _():
        _vq_body(i, n_steps, f, e, mmb_ref, mma_ref, *common)


_sc_mesh = plsc.VectorSubcoreMesh(core_axis_name="c", subcore_axis_name="s")


@functools.partial(
    pl.kernel,
    mesh=_sc_mesh,
    out_type=jax.ShapeDtypeStruct((N, D), jnp.float32),
    scratch_types=[
        pltpu.VMEM((SC_NW, ROWS_PER_W // GCHUNK, GCHUNK), jnp.int32),
        pltpu.VMEM((ROWS_PER_W, D), jnp.float32),
        pltpu.SemaphoreType.DMA,
    ],
)
def _sc_gather(emb_hbm, idx_hbm, out_hbm, idx_v, rows_v, sem):
    wid = lax.axis_index("s") * SC_NC + lax.axis_index("c")
    base = wid * ROWS_PER_W
    idx_w = idx_v.at[wid]
    pltpu.sync_copy(idx_hbm.at[wid], idx_w)
    cps = [
        pltpu.async_copy(emb_hbm.at[idx_w.at[c]],
                         rows_v.at[pl.ds(c * GCHUNK, GCHUNK)], sem)
        for c in range(ROWS_PER_W // GCHUNK)
    ]
    for cp in cps:
        cp.wait()
    pltpu.sync_copy(rows_v, out_hbm.at[pl.ds(base, ROWS_PER_W)])


@jax.jit
def kernel(z_e, embedding):
    z = jnp.transpose(z_e, (0, 2, 3, 1))
    flat = z.reshape(-1, D)
    f_sq = jnp.sum(flat ** 2, axis=1, keepdims=True)
    e_sq = jnp.sum(embedding ** 2, axis=1)[None, :]

    n_tiles = N // NT
    enc, idx, loss, perp = pl.pallas_call(
        _vq_kernel,
        grid=(n_tiles + 1,),
        in_specs=[
            pl.BlockSpec((NT, D), lambda i: (jnp.minimum(i, n_tiles - 1), 0)),
            pl.BlockSpec((K, D), lambda i: (0, 0)),
            pl.BlockSpec((NT, 1), lambda i: (jnp.maximum(i - 1, 0), 0)),
            pl.BlockSpec((1, K), lambda i: (0, 0)),
        ],
        out_specs=[
            pl.BlockSpec((NT, K), lambda i: (jnp.maximum(i - 1, 0), 0)),
            pl.BlockSpec((NT, 1), lambda i: (jnp.maximum(i - 1, 0), 0)),
            pl.BlockSpec((1, 1), lambda i: (0, 0)),
            pl.BlockSpec((1, 1), lambda i: (0, 0)),
        ],
        out_shape=[
            jax.ShapeDtypeStruct((N, K), jnp.float32),
            jax.ShapeDtypeStruct((N, 1), jnp.int32),
            jax.ShapeDtypeStruct((1, 1), jnp.float32),
            jax.ShapeDtypeStruct((1, 1), jnp.float32),
        ],
        scratch_shapes=[
            pltpu.VMEM((NT, K), jnp.float32),
            pltpu.VMEM((NT, K), jnp.float32),
            pltpu.VMEM((1, K), jnp.float32),
            pltpu.SMEM((1,), jnp.float32),
        ],
    )(flat, embedding, f_sq, e_sq)

    idx3 = idx.reshape(SC_NW, ROWS_PER_W // GCHUNK, GCHUNK)
    q = _sc_gather(embedding, idx3)

    q_st = flat + (q - flat)
    quantized = jnp.transpose(q_st.reshape(z.shape), (0, 3, 1, 2))
    return quantized, loss[0, 0], perp[0, 0], enc


# speedup vs baseline: 1.2080x; 1.2080x over previous
import functools

import jax
import jax.numpy as jnp
from jax import lax
from jax.experimental import pallas as pl
from jax.experimental.pallas import tpu as pltpu
from jax.experimental.pallas import tpu_sc as plsc

K = 8192
D = 256
BETA = 0.25
NT = 512
N = 8192

SC_NC = 2
SC_NS = 16
SC_NW = SC_NC * SC_NS
ROWS_PER_W = N // SC_NW
GCHUNK = 128


def _vq_kernel(flat_ref, emb_ref, fsq_ref, esq_ref, enc_ref, idx_ref,
               loss_ref, perp_ref, counts_ref, loss_acc_ref):
    i = pl.program_id(0)
    n_tiles = pl.num_programs(0)
    f = flat_ref[:]
    e = emb_ref[:]

    f_sq = fsq_ref[:]
    e_sq = esq_ref[:]
    mm = jnp.dot(f * (-2.0), e.T, preferred_element_type=jnp.float32)
    d = (f_sq + e_sq) + mm

    m = jnp.min(d, axis=1, keepdims=True)
    iota = jax.lax.broadcasted_iota(jnp.int32, (NT, K), 1).astype(jnp.float32)
    t = jnp.where(d == m, iota, jnp.float32(K))
    idx_f = jnp.min(t, axis=1, keepdims=True)
    idx_ref[:] = idx_f.astype(jnp.int32)

    enc = (iota == idx_f).astype(jnp.float32)
    enc_ref[:] = enc

    @pl.when(i == 0)
    def _():
        counts_ref[:] = jnp.zeros_like(counts_ref)
        loss_acc_ref[0] = 0.0

    ones = jnp.ones((8, NT), jnp.float32)
    counts_ref[:] += jnp.dot(ones, enc,
                             preferred_element_type=jnp.float32)[0:1, :]
    loss_acc_ref[0] += jnp.sum(m)

    @pl.when(i == n_tiles - 1)
    def _():
        p = counts_ref[0, :] * (1.0 / N)
        perp = jnp.exp(-jnp.sum(p * jnp.log(p + 1e-10)))
        perp_ref[:] = perp.reshape(1, 1)
        loss_ref[:] = ((1.0 + BETA) * loss_acc_ref[0] / (N * D)).reshape(1, 1)


_sc_mesh = plsc.VectorSubcoreMesh(core_axis_name="c", subcore_axis_name="s")


@functools.partial(
    pl.kernel,
    mesh=_sc_mesh,
    out_type=jax.ShapeDtypeStruct((N, D), jnp.float32),
    scratch_types=[
        pltpu.VMEM((SC_NW, ROWS_PER_W // GCHUNK, GCHUNK), jnp.int32),
        pltpu.VMEM((ROWS_PER_W, D), jnp.float32),
        pltpu.SemaphoreType.DMA,
    ],
)
def _sc_gather(emb_hbm, idx_hbm, out_hbm, idx_v, rows_v, sem):
    wid = lax.axis_index("s") * SC_NC + lax.axis_index("c")
    base = wid * ROWS_PER_W
    idx_w = idx_v.at[wid]
    pltpu.sync_copy(idx_hbm.at[wid], idx_w)
    cps = [
        pltpu.async_copy(emb_hbm.at[idx_w.at[c]],
                         rows_v.at[pl.ds(c * GCHUNK, GCHUNK)], sem)
        for c in range(ROWS_PER_W // GCHUNK)
    ]
    for cp in cps:
        cp.wait()
    pltpu.sync_copy(rows_v, out_hbm.at[pl.ds(base, ROWS_PER_W)])


@jax.jit
def kernel(z_e, embedding):
    z = jnp.transpose(z_e, (0, 2, 3, 1))
    flat = z.reshape(-1, D)
    f_sq = jnp.sum(flat ** 2, axis=1, keepdims=True)
    e_sq = jnp.sum(embedding ** 2, axis=1)[None, :]

    enc, idx, loss, perp = pl.pallas_call(
        _vq_kernel,
        grid=(N // NT,),
        in_specs=[
            pl.BlockSpec((NT, D), lambda i: (i, 0)),
            pl.BlockSpec((K, D), lambda i: (0, 0)),
            pl.BlockSpec((NT, 1), lambda i: (i, 0)),
            pl.BlockSpec((1, K), lambda i: (0, 0)),
        ],
        out_specs=[
            pl.BlockSpec((NT, K), lambda i: (i, 0)),
            pl.BlockSpec((NT, 1), lambda i: (i, 0)),
            pl.BlockSpec((1, 1), lambda i: (0, 0)),
            pl.BlockSpec((1, 1), lambda i: (0, 0)),
        ],
        out_shape=[
            jax.ShapeDtypeStruct((N, K), jnp.float32),
            jax.ShapeDtypeStruct((N, 1), jnp.int32),
            jax.ShapeDtypeStruct((1, 1), jnp.float32),
            jax.ShapeDtypeStruct((1, 1), jnp.float32),
        ],
        scratch_shapes=[
            pltpu.VMEM((1, K), jnp.float32),
            pltpu.SMEM((1,), jnp.float32),
        ],
    )(flat, embedding, f_sq, e_sq)

    idx3 = idx.reshape(SC_NW, ROWS_PER_W // GCHUNK, GCHUNK)
    q = _sc_gather(embedding, idx3)

    q_st = flat + (q - flat)
    quantized = jnp.transpose(q_st.reshape(z.shape), (0, 3, 1, 2))
    return quantized, loss[0, 0], perp[0, 0], enc
